# baseline (speedup 1.0000x reference)
import functools

import jax
import jax.numpy as jnp
from jax import lax
from jax.experimental import pallas as pl
from jax.experimental.pallas import tpu as pltpu
from jax.experimental.pallas import tpu_sc as plsc

N_NODES = 10000
D_FEAT = 128
N_EDGES = 320000

NC = 2
NS = 16
NW = NC * NS

EDGE_B = 128
BATCH_PER_W = 80
N_BATCH = NW * BATCH_PER_W
E_PAD = N_BATCH * EDGE_B
NBUF = 2
N_ITER = BATCH_PER_W // NBUF
ACC_ROWS = N_NODES + EDGE_B

ROW_CHUNK = 200
N_CHUNK = N_NODES // ROW_CHUNK
CHUNK_PER_S = -(-N_CHUNK // NS)


def _sc_partial(x, src1d, dst1d, zrows):
    mesh = plsc.VectorSubcoreMesh(core_axis_name="c", subcore_axis_name="s")

    scratch = dict(
        acc=pltpu.VMEM_SHARED((ACC_ROWS, D_FEAT), jnp.float32),
    )
    for b in range(NBUF):
        scratch[f"rows{b}"] = pltpu.VMEM((EDGE_B, D_FEAT), jnp.float32)
        scratch[f"gsem{b}"] = pltpu.SemaphoreType.DMA
        scratch[f"ssem{b}"] = pltpu.SemaphoreType.DMA
        for p in range(2):
            scratch[f"sidx{p}{b}"] = pltpu.VMEM((EDGE_B,), jnp.int32)
            scratch[f"didx{p}{b}"] = pltpu.VMEM((EDGE_B,), jnp.int32)
            scratch[f"isem{p}{b}"] = pltpu.SemaphoreType.DMA
            scratch[f"dsem{p}{b}"] = pltpu.SemaphoreType.DMA

    @functools.partial(
        pl.kernel,
        out_type=jax.ShapeDtypeStruct((NC, N_NODES, D_FEAT), jnp.float32),
        mesh=mesh,
        scratch_types=scratch,
    )
    def kern(x_hbm, s_hbm, d_hbm, z_hbm, part_hbm, *, acc, **bufs):
        rows = [bufs[f"rows{b}"] for b in range(NBUF)]
        gsem = [bufs[f"gsem{b}"] for b in range(NBUF)]
        ssem = [bufs[f"ssem{b}"] for b in range(NBUF)]
        sidx = [[bufs[f"sidx{p}{b}"] for b in range(NBUF)] for p in range(2)]
        didx = [[bufs[f"didx{p}{b}"] for b in range(NBUF)] for p in range(2)]
        isem = [[bufs[f"isem{p}{b}"] for b in range(NBUF)] for p in range(2)]
        dsem = [[bufs[f"dsem{p}{b}"] for b in range(NBUF)] for p in range(2)]

        c = lax.axis_index("c")
        s = lax.axis_index("s")
        w = c * NS + s
        b0 = w * BATCH_PER_W

        def fire_idx(t, p):
            for b in range(NBUF):
                e0 = (b0 + t * NBUF + b) * EDGE_B
                pltpu.async_copy(s_hbm.at[pl.ds(e0, EDGE_B)], sidx[p][b], isem[p][b])
                pltpu.async_copy(d_hbm.at[pl.ds(e0, EDGE_B)], didx[p][b], dsem[p][b])

        def wait_idx(p):
            for b in range(NBUF):
                pltpu.make_async_copy(
                    s_hbm.at[pl.ds(0, EDGE_B)], sidx[p][b], isem[p][b]
                ).wait()
                pltpu.make_async_copy(
                    d_hbm.at[pl.ds(0, EDGE_B)], didx[p][b], dsem[p][b]
                ).wait()

        def scat_wait(b):
            pltpu.make_async_copy(
                x_hbm.at[pl.ds(0, EDGE_B), :], rows[b], ssem[b]
            ).wait()

        fire_idx(0, 0)

        def zchunk(i, _):
            ch = s + i * NS

            @pl.when(ch < N_CHUNK)
            def _():
                pltpu.sync_copy(z_hbm, acc.at[pl.ds(ch * ROW_CHUNK, ROW_CHUNK), :])
            return 0

        lax.fori_loop(0, CHUNK_PER_S, zchunk, 0)

        @pl.when(s == 0)
        def _():
            pltpu.sync_copy(
                z_hbm.at[pl.ds(0, EDGE_B), :], acc.at[pl.ds(N_NODES, EDGE_B), :]
            )

        plsc.subcore_barrier()

        def half(u, p):
            @pl.when(u > 0)
            def _():
                for b in range(NBUF):
                    scat_wait(b)
            wait_idx(p)

            @pl.when(u + 1 < N_ITER)
            def _():
                fire_idx(u + 1, 1 - p)

            gets = [
                pltpu.async_copy(x_hbm.at[sidx[p][b]], rows[b], gsem[b])
                for b in range(NBUF)
            ]
            for b in range(NBUF):
                gets[b].wait()
                pltpu.async_copy(rows[b], acc.at[didx[p][b]], ssem[b], add=True)

        def outer(v, _):
            half(2 * v, 0)
            half(2 * v + 1, 1)
            return 0

        lax.fori_loop(0, N_ITER // 2, outer, 0)
        for b in range(NBUF):
            scat_wait(b)
        plsc.subcore_barrier()

        def wchunk(i, _):
            ch = s + i * NS

            @pl.when(ch < N_CHUNK)
            def _():
                r0 = ch * ROW_CHUNK
                pltpu.sync_copy(
                    acc.at[pl.ds(r0, ROW_CHUNK), :],
                    part_hbm.at[c, pl.ds(r0, ROW_CHUNK), :],
                )
            return 0

        lax.fori_loop(0, CHUNK_PER_S, wchunk, 0)

    return kern(x, src1d, dst1d, zrows)


def _combine(parts):
    blk = 400

    def body(p_ref, o_ref):
        o_ref[...] = p_ref[0] + p_ref[1]

    return pl.pallas_call(
        body,
        grid=(N_NODES // blk,),
        in_specs=[pl.BlockSpec((NC, blk, D_FEAT), lambda i: (0, i, 0))],
        out_specs=pl.BlockSpec((blk, D_FEAT), lambda i: (i, 0)),
        out_shape=jax.ShapeDtypeStruct((N_NODES, D_FEAT), jnp.float32),
    )(parts)


def kernel(x, edge_index):
    ei = edge_index.astype(jnp.int32)
    n_pad = E_PAD - N_EDGES
    src1d = jnp.concatenate([ei[0], jnp.zeros((n_pad,), jnp.int32)])
    dst1d = jnp.concatenate([ei[1], jnp.full((n_pad,), N_NODES, jnp.int32)])
    zrows = jnp.zeros((ROW_CHUNK, D_FEAT), jnp.float32)
    parts = _sc_partial(x, src1d, dst1d, zrows)
    return _combine(parts)
